# baseline (speedup 1.0000x reference)
import jax
import jax.numpy as jnp
from jax.experimental import pallas as pl
from jax.experimental.pallas import tpu as pltpu

_NUM_GROUPS = 64
_EPS = 1e-5

_MISH_COEFS = (
    0.6082008481025696,
    0.3327913284301758,
    -0.011037762276828289,
    -0.04728490114212036,
    -0.0013827549992129207,
    0.005855408962816,
)


def _clamped_mish_poly(c):
    acc = jnp.full_like(c, _MISH_COEFS[-1])
    for coef in _MISH_COEFS[-2::-1]:
        acc = acc * c + coef
    return acc * c


def _fused_body(x_ref, b_ref, bt_ref, mr_ref, mt_ref, mtg_ref, o_ref):
    t = jnp.clip(x_ref[...] + b_ref[...], -1.0, 1.0)
    v = _clamped_mish_poly(t)
    q = v * v

    mr = mr_ref[...]
    mean = jnp.dot(v, mr, preferred_element_type=jnp.float32)
    msq = jnp.dot(q, mr, preferred_element_type=jnp.float32)

    var = jnp.maximum(msq - mean * mean, 0.0)
    inv_std = jax.lax.rsqrt(var + _EPS)

    mean_b = jnp.dot(mean, mt_ref[...], preferred_element_type=jnp.float32)
    scale_b = jnp.dot(inv_std, mtg_ref[...],
                      preferred_element_type=jnp.float32)

    o_ref[...] = (v - mean_b) * scale_b + bt_ref[...]


def kernel(x, bias, gamma, beta):
    n, c = x.shape
    g = _NUM_GROUPS
    gs = c // g
    block_n = 256

    chan = jnp.arange(c, dtype=jnp.int32) // gs
    onehot = (chan[:, None]
              == jnp.arange(g, dtype=jnp.int32)[None, :]).astype(jnp.float32)
    mr = onehot * (1.0 / gs)
    mt = onehot.T
    mtg = mt * gamma[None, :]

    grid = (n // block_n,)
    row_spec = pl.BlockSpec((block_n, c), lambda i: (i, 0))
    param_spec = lambda shape: pl.BlockSpec(shape, lambda i: (0, 0))

    return pl.pallas_call(
        _fused_body,
        grid=grid,
        in_specs=[
            row_spec,
            param_spec((1, c)),
            param_spec((1, c)),
            param_spec((c, g)),
            param_spec((g, c)),
            param_spec((g, c)),
        ],
        out_specs=row_spec,
        out_shape=jax.ShapeDtypeStruct((n, c), jnp.float32),
        compiler_params=pltpu.CompilerParams(
            dimension_semantics=("parallel",),
        ),
    )(x, bias.reshape(1, c), beta.reshape(1, c), mr, mt, mtg)


# speedup vs baseline: 1.1778x; 1.1778x over previous
import jax
import jax.numpy as jnp
from jax.experimental import pallas as pl
from jax.experimental.pallas import tpu as pltpu

_NUM_GROUPS = 64
_EPS = 1e-5

_MISH_COEFS = (
    0.6082008481025696,
    0.3327913284301758,
    -0.011037762276828289,
    -0.04728490114212036,
    -0.0013827549992129207,
    0.005855408962816,
)


def _clamped_mish_poly(c):
    acc = jnp.full_like(c, _MISH_COEFS[-1])
    for coef in _MISH_COEFS[-2::-1]:
        acc = acc * c + coef
    return acc * c


def _fused_body(x_ref, b_ref, bt_ref, mr_ref, mt_ref, mtg_ref, o_ref):
    t = jnp.clip(x_ref[...] + b_ref[...], -1.0, 1.0)
    v = _clamped_mish_poly(t)
    q = v * v

    mr = mr_ref[...]
    mean = jnp.dot(v, mr, preferred_element_type=jnp.float32)
    msq = jnp.dot(q, mr, preferred_element_type=jnp.float32)

    var = jnp.maximum(msq - mean * mean, 0.0)
    inv_std = jax.lax.rsqrt(var + _EPS)

    mean_b = jnp.dot(mean, mt_ref[...], preferred_element_type=jnp.float32)
    scale_b = jnp.dot(inv_std, mtg_ref[...],
                      preferred_element_type=jnp.float32)

    o_ref[...] = (v - mean_b) * scale_b + bt_ref[...]


def kernel(x, bias, gamma, beta):
    n, c = x.shape
    g = _NUM_GROUPS
    gs = c // g
    block_n = 512

    chan = jnp.arange(c, dtype=jnp.int32) // gs
    onehot = (chan[:, None]
              == jnp.arange(g, dtype=jnp.int32)[None, :]).astype(jnp.float32)
    mr = onehot * (1.0 / gs)
    mt = onehot.T
    mtg = mt * gamma[None, :]

    grid = (n // block_n,)
    row_spec = pl.BlockSpec((block_n, c), lambda i: (i, 0))
    param_spec = lambda shape: pl.BlockSpec(shape, lambda i: (0, 0))

    return pl.pallas_call(
        _fused_body,
        grid=grid,
        in_specs=[
            row_spec,
            param_spec((1, c)),
            param_spec((1, c)),
            param_spec((c, g)),
            param_spec((g, c)),
            param_spec((g, c)),
        ],
        out_specs=row_spec,
        out_shape=jax.ShapeDtypeStruct((n, c), jnp.float32),
        compiler_params=pltpu.CompilerParams(
            dimension_semantics=("parallel",),
        ),
    )(x, bias.reshape(1, c), beta.reshape(1, c), mr, mt, mtg)


# speedup vs baseline: 1.2810x; 1.0876x over previous
import jax
import jax.numpy as jnp
from jax.experimental import pallas as pl
from jax.experimental.pallas import tpu as pltpu

_NUM_GROUPS = 64
_EPS = 1e-5

_MISH_COEFS = (
    0.6082008481025696,
    0.3327913284301758,
    -0.011037762276828289,
    -0.04728490114212036,
    -0.0013827549992129207,
    0.005855408962816,
)


def _clamped_mish_poly(c):
    acc = jnp.full_like(c, _MISH_COEFS[-1])
    for coef in _MISH_COEFS[-2::-1]:
        acc = acc * c + coef
    return acc * c


def _fused_body(x_ref, b_ref, bt_ref, mr_ref, mt_ref, mtg_ref, o_ref):
    t = jnp.clip(x_ref[...] + b_ref[...], -1.0, 1.0)
    v = _clamped_mish_poly(t)
    q = v * v

    mr = mr_ref[...]
    mean = jnp.dot(v, mr, preferred_element_type=jnp.float32)
    msq = jnp.dot(q, mr, preferred_element_type=jnp.float32)

    var = jnp.maximum(msq - mean * mean, 0.0)
    inv_std = jax.lax.rsqrt(var + _EPS)

    mean_b = jnp.dot(mean, mt_ref[...], preferred_element_type=jnp.float32)
    scale_b = jnp.dot(inv_std, mtg_ref[...],
                      preferred_element_type=jnp.float32)

    o_ref[...] = (v - mean_b) * scale_b + bt_ref[...]


def kernel(x, bias, gamma, beta):
    n, c = x.shape
    g = _NUM_GROUPS
    gs = c // g
    block_n = 1024

    chan = jnp.arange(c, dtype=jnp.int32) // gs
    onehot = (chan[:, None]
              == jnp.arange(g, dtype=jnp.int32)[None, :]).astype(jnp.float32)
    mr = onehot * (1.0 / gs)
    mt = onehot.T
    mtg = mt * gamma[None, :]

    grid = (n // block_n,)
    row_spec = pl.BlockSpec((block_n, c), lambda i: (i, 0))
    param_spec = lambda shape: pl.BlockSpec(shape, lambda i: (0, 0))

    return pl.pallas_call(
        _fused_body,
        grid=grid,
        in_specs=[
            row_spec,
            param_spec((1, c)),
            param_spec((1, c)),
            param_spec((c, g)),
            param_spec((g, c)),
            param_spec((g, c)),
        ],
        out_specs=row_spec,
        out_shape=jax.ShapeDtypeStruct((n, c), jnp.float32),
        compiler_params=pltpu.CompilerParams(
            dimension_semantics=("parallel",),
        ),
    )(x, bias.reshape(1, c), beta.reshape(1, c), mr, mt, mtg)
